# baseline (speedup 1.0000x reference)
import functools

import jax
import jax.numpy as jnp
from jax.experimental import pallas as pl
from jax.experimental.pallas import tpu as pltpu

F32 = jnp.float32
BF16 = jnp.bfloat16
F8 = jnp.float8_e4m3fn
SE = 1024.0
ST = 4096.0


def _mega_kernel(n1, T, L, n_rel, E, N, Np, bN0, P0, bN, CH,
                 ids_ref, w3_ref, wt_ref, x_ref, t2rT_ref,
                 e2f_ref, t2ef_ref, t2tq_ref,
                 out_ref,
                 e2q_ref, t2eq_ref, pstack_ref,
                 x48_ref, wsel_ref, sacc_ref, scur_ref, c_ref):
    i = pl.program_id(0)
    B = x_ref.shape[0]
    BL = B * L
    m = t2tq_ref.shape[1]

    @pl.when(i == 0)
    def _prep():
        rc = jax.lax.broadcasted_iota(jnp.int32, (B, n1), 1)
        oh = (ids_ref[:, 0:1] == rc).astype(F32)
        ri = jax.lax.broadcasted_iota(jnp.int32, (BL, B), 0)
        ci = jax.lax.broadcasted_iota(jnp.int32, (BL, B), 1)
        ebc = ((ri // 3) == ci).astype(F32)

        x48_ref[...] = jnp.dot(ebc.astype(BF16), x_ref[...],
                               preferred_element_type=F32).astype(BF16)
        for t in range(T):
            pstack = jnp.zeros((BL, n_rel), dtype=F32)
            for l in range(L):
                g = jnp.dot(oh.astype(BF16), w3_ref[t * L + l], preferred_element_type=F32)
                g = g - jnp.max(g, axis=1, keepdims=True)
                p = jnp.exp(g)
                p = p / jnp.sum(p, axis=1, keepdims=True)
                el = ((ri - 3 * (ri // 3)) == l) & ((ri // 3) == ci)
                pstack = pstack + jnp.dot(el.astype(F32), p,
                                          preferred_element_type=F32)
            pstack_ref[t * BL:(t + 1) * BL, :] = pstack.astype(BF16)
        wts = jax.nn.sigmoid(jnp.dot(oh, wt_ref[...],
                                     preferred_element_type=F32))
        ri2 = jax.lax.broadcasted_iota(jnp.int32, (B, BL), 0)
        ci2 = jax.lax.broadcasted_iota(jnp.int32, (B, BL), 1)
        tile = ((ci2 - 3 * (ci2 // 3))[:L, :] ==
                jax.lax.broadcasted_iota(jnp.int32, (L, BL), 0)).astype(F32)
        wtile = jnp.dot(wts, tile, preferred_element_type=F32)
        wsel_ref[...] = wtile * ((ci2 // 3) == ri2).astype(F32)
        sacc_ref[...] = jnp.zeros_like(sacc_ref)

    def srel_chunk(base, width, srow):
        ps = pstack_ref[srow:srow + BL, :]
        sr = jnp.dot(ps, t2rT_ref[:, pl.ds(base, width)],
                     preferred_element_type=F32)
        colb = base + jax.lax.broadcasted_iota(jnp.int32, (B, width), 1)
        spanb = (((ids_ref[:, 1:2] <= colb) & (colb < ids_ref[:, 2:3])) |
                 (colb < E)).astype(BF16)
        rie = jax.lax.broadcasted_iota(jnp.int32, (BL, B), 0)
        cie = jax.lax.broadcasted_iota(jnp.int32, (BL, B), 1)
        ebce = ((rie // 3) == cie).astype(BF16)
        span = jnp.dot(ebce, spanb, preferred_element_type=F32)
        return sr * span

    @pl.when(i < P0)
    def _load_quant_hop0():
        sl = pl.ds(i * bN0, bN0)

        @pl.when(i < P0 - 1)
        def _full_slab():
            e2q_ref[:, sl] = (e2f_ref[...] * SE).astype(F8)
            t2eq_ref[sl, :] = (t2ef_ref[...] * ST).astype(F8)

        @pl.when(i == P0 - 1)
        def _partial_slab():
            colv = (i * bN0 +
                    jax.lax.broadcasted_iota(jnp.int32, e2f_ref.shape, 1))
            e2q_ref[:, sl] = jnp.where(colv < N, e2f_ref[...] * SE,
                                       0.0).astype(F8)
            rowv = (i * bN0 +
                    jax.lax.broadcasted_iota(jnp.int32, t2ef_ref.shape, 0))
            t2eq_ref[sl, :] = jnp.where(rowv < N, t2ef_ref[...] * ST,
                                        0.0).astype(F8)

        e2q = e2q_ref[:, sl]
        t2eq = t2eq_ref[sl, :]
        xo = jnp.dot(x48_ref[...], e2q, preferred_element_type=F32) * (1.0 / SE)
        masked = (xo * srel_chunk(i * bN0, bN0, 0)).astype(BF16)
        sacc_ref[...] += jnp.dot(masked, t2eq, preferred_element_type=F32)

    @pl.when(i == P0 - 1)
    def _finish_hop0():
        scur_ref[...] = jnp.clip(sacc_ref[...] * (1.0 / ST),
                                 0.0, 1.0).astype(BF16)

    def phase_a(j):
        sl = pl.ds(j * bN, bN)
        sp = scur_ref[...]
        xp = jax.lax.dot_general(sp, t2eq_ref[sl, :],
                                 (((1,), (1,)), ((), ())),
                                 preferred_element_type=F32)
        sgn = ((xp > 0.0).astype(F32) - (xp < 0.0).astype(F32)).astype(BF16)
        contrib = jnp.dot(sgn, t2tq_ref[...], preferred_element_type=F32)

        @pl.when(j == 0)
        def _():
            c_ref[...] = jnp.zeros_like(c_ref)

        c_ref[...] += contrib

    def phase_b(j, srow, final):
        sl = pl.ds(j * bN, bN)
        sp = scur_ref[...]
        x = jnp.dot(sp, e2q_ref[:, sl],
                    preferred_element_type=F32) * (1.0 / SE)
        ti = jax.lax.broadcasted_iota(jnp.int32, (m, m), 0)
        tj = jax.lax.broadcasted_iota(jnp.int32, (m, m), 1)
        triu = (ti <= tj).astype(F32)
        c2 = jnp.dot(c_ref[...], triu, preferred_element_type=F32).astype(BF16)
        mv = jax.lax.dot_general(c2, t2tq_ref[...],
                                 (((1,), (1,)), ((), ())),
                                 preferred_element_type=F32)
        mv = jnp.clip(mv, 0.0, 1.0)
        col = j * bN + jax.lax.broadcasted_iota(jnp.int32, x.shape, 1)
        mv = jnp.where(col < E, 1.0, mv)
        masked = (x * mv * srel_chunk(j * bN, bN, srow)).astype(BF16)
        contrib = jnp.dot(masked, t2eq_ref[sl, :], preferred_element_type=F32)

        @pl.when(j == 0)
        def _():
            sacc_ref[...] = jnp.zeros_like(sacc_ref)

        sacc_ref[...] += contrib

        @pl.when(j == CH - 1)
        def _():
            sc = jnp.clip(sacc_ref[...] * (1.0 / ST), 0.0, 1.0)
            scur_ref[...] = sc.astype(BF16)
            if final:
                out_ref[...] = jnp.dot(wsel_ref[...], sc,
                                       precision=jax.lax.Precision.HIGHEST,
                                       preferred_element_type=F32)

    @pl.when((i >= P0) & (i < P0 + CH))
    def _a1():
        phase_a(i - P0)

    @pl.when((i >= P0 + CH) & (i < P0 + 2 * CH))
    def _b1():
        phase_b(i - (P0 + CH), BL, False)

    @pl.when((i >= P0 + 2 * CH) & (i < P0 + 3 * CH))
    def _a2():
        phase_a(i - (P0 + 2 * CH))

    @pl.when((i >= P0 + 3 * CH) & (i < P0 + 4 * CH))
    def _b2():
        phase_b(i - (P0 + 3 * CH), 2 * BL, True)


def kernel(input_x, input_r, input_triple2id, e2triple, triple2e, triple2r,
           triple2time, w_params, weight_params):
    B, E = input_x.shape
    N = triple2e.shape[0]
    m = triple2time.shape[1]
    n1, T, L, n_rel = w_params.shape
    BL = B * L

    bN0 = 512
    P0 = -(-N // bN0)
    Np = P0 * bN0
    bN = 2048
    CH = Np // bN
    nsteps = P0 + 4 * CH

    t2tq = jnp.pad(triple2time, ((0, Np - N), (0, 0))).astype(F8)
    t2rT = jnp.pad(triple2r.T, ((0, 0), (0, Np - N))).astype(BF16)
    w3 = jnp.transpose(w_params.reshape(n1, T * L, n_rel), (1, 0, 2)).astype(BF16)
    wt2 = weight_params[..., 0]
    ids = jnp.concatenate([
        input_r.astype(jnp.int32).reshape(B, 1),
        input_triple2id.astype(jnp.int32),
        jnp.zeros((B, 1), jnp.int32)], axis=1)

    c0 = lambda i: (0, 0)
    out = pl.pallas_call(
        functools.partial(_mega_kernel, n1, T, L, n_rel, E, N, Np,
                          bN0, P0, bN, CH),
        grid=(nsteps,),
        in_specs=[
            pl.BlockSpec((B, 4), c0),
            pl.BlockSpec((T * L, n1, n_rel), lambda i: (0, 0, 0)),
            pl.BlockSpec((n1, L), c0),
            pl.BlockSpec((B, E), c0),
            pl.BlockSpec((n_rel, Np), c0),
            pl.BlockSpec((E, bN0), lambda i: (0, jnp.minimum(i, P0 - 1))),
            pl.BlockSpec((bN0, E), lambda i: (jnp.minimum(i, P0 - 1), 0)),
            pl.BlockSpec((bN, m),
                         lambda i: (jnp.where(i < P0, 0,
                                              jax.lax.rem(i - P0, CH)), 0)),
        ],
        out_specs=pl.BlockSpec((B, E), c0),
        out_shape=jax.ShapeDtypeStruct((B, E), F32),
        scratch_shapes=[
            pltpu.VMEM((E, Np), F8),
            pltpu.VMEM((Np, E), F8),
            pltpu.VMEM((T * BL, n_rel), BF16),
            pltpu.VMEM((BL, E), BF16),
            pltpu.VMEM((B, BL), F32),
            pltpu.VMEM((BL, E), F32),
            pltpu.VMEM((BL, E), BF16),
            pltpu.VMEM((BL, m), F32),
        ],
    )(ids, w3, wt2, input_x.astype(BF16), t2rT,
      e2triple, triple2e, t2tq)
    return out
